# baseline (speedup 1.0000x reference)
import functools

import jax
import jax.numpy as jnp
from jax.experimental import pallas as pl
from jax.experimental.pallas import tpu as pltpu

B = 4096
H = 128
CAP_E = 4096
CAP_S = 16384
TOPK_E = 8
TOPK_S = 4
RECENCY_DECAY = 0.99
TIMESTAMP = 100.0

BQ = 128


def _l2n(x):
    n = jnp.sqrt(jnp.sum(x * x, axis=-1, keepdims=True))
    return x / jnp.maximum(n, 1e-12)


def _prep_ep_kernel(store_ref, imp_ref, ts_ref, wk_ref, wv_ref,
                    kn_ref, vals_ref, w_ref):
    store = store_ref[...]
    keys = jnp.dot(store, wk_ref[...].T, preferred_element_type=jnp.float32)
    kn_ref[...] = _l2n(keys)
    ages = TIMESTAMP - ts_ref[...]
    recency = jnp.exp(-jnp.abs(ages) * (1.0 - RECENCY_DECAY))
    w = recency * (imp_ref[...] + 1.0)
    w_ref[...] = w / (jnp.sum(w) + 1e-8)
    vals_ref[...] = jnp.dot(store, wv_ref[...].T,
                            preferred_element_type=jnp.float32)


def _prep_sem_kernel(keys_ref, wk_ref, out_ref):
    k = jnp.dot(keys_ref[...], wk_ref[...].T,
                preferred_element_type=jnp.float32)
    out_ref[...] = _l2n(k)


def _rank_thresh(sim, k):
    t = jnp.max(sim, axis=-1, keepdims=True)
    for _ in range(k - 1):
        t = jnp.max(jnp.where(sim < t, sim, -jnp.inf),
                    axis=-1, keepdims=True)
    return t


def _masked_attn(sim, k):
    rowmax = jnp.max(sim, axis=-1, keepdims=True)
    t = rowmax
    for _ in range(k - 1):
        t = jnp.max(jnp.where(sim < t, sim, -jnp.inf),
                    axis=-1, keepdims=True)
    p = jnp.where(sim >= t, jnp.exp(sim - rowmax), 0.0)
    return p / jnp.sum(p, axis=-1, keepdims=True)


def _main_kernel(q_ref, kn_ref, vals_ref, w_ref, ksn_ref, semv_ref,
                 sem_wq_ref, ep_wo_ref, sem_wo_ref,
                 g_w1_ref, g_b1_ref, g_w2_ref, g_b2_ref, r_wo_ref,
                 out_ref):
    q = q_ref[...]
    qn = _l2n(q)

    sim_e = jnp.dot(qn, kn_ref[...].T, preferred_element_type=jnp.float32)
    sim_e = sim_e * w_ref[...]
    we = _masked_attn(sim_e, TOPK_E)
    ep_out = jnp.dot(we, vals_ref[...], preferred_element_type=jnp.float32)
    ep_out = jnp.dot(ep_out, ep_wo_ref[...].T,
                     preferred_element_type=jnp.float32)

    qs = jnp.dot(q, sem_wq_ref[...].T, preferred_element_type=jnp.float32)
    qsn = _l2n(qs)
    sim_s = jnp.dot(qsn, ksn_ref[...].T, preferred_element_type=jnp.float32)
    ws = _masked_attn(sim_s, TOPK_S)
    sem_out = jnp.dot(ws, semv_ref[...], preferred_element_type=jnp.float32)
    sem_out = jnp.dot(sem_out, sem_wo_ref[...].T,
                      preferred_element_type=jnp.float32)

    h = jnp.dot(q, g_w1_ref[...].T, preferred_element_type=jnp.float32)
    h = h + g_b1_ref[...]
    h = h * jax.nn.sigmoid(h)
    logits = jnp.dot(h, g_w2_ref[...].T, preferred_element_type=jnp.float32)
    logits = logits + g_b2_ref[...]
    col = jax.lax.broadcasted_iota(jnp.int32, logits.shape, 1)
    logits = jnp.where(col < 3, logits, -jnp.inf)
    gmax = jnp.max(logits, axis=-1, keepdims=True)
    ge = jnp.exp(logits - gmax)
    gw = ge / jnp.sum(ge, axis=-1, keepdims=True)

    blended = (gw[:, 0:1] * q + gw[:, 1:2] * ep_out + gw[:, 2:3] * sem_out)
    out_ref[...] = jnp.dot(blended, r_wo_ref[...].T,
                           preferred_element_type=jnp.float32)


@jax.jit
def kernel(query, ep_store, ep_importance, ep_timestamps, ep_Wk, ep_Wv, ep_Wo,
           sem_keys, sem_values, sem_Wq, sem_Wk, sem_Wo,
           gate_W1, gate_b1, gate_W2, gate_b2, router_Wo):
    f32 = jnp.float32

    kn, vals, w_row = pl.pallas_call(
        _prep_ep_kernel,
        out_shape=(jax.ShapeDtypeStruct((CAP_E, H), f32),
                   jax.ShapeDtypeStruct((CAP_E, H), f32),
                   jax.ShapeDtypeStruct((1, CAP_E), f32)),
    )(ep_store, ep_importance.reshape(1, CAP_E),
      ep_timestamps.reshape(1, CAP_E), ep_Wk, ep_Wv)

    ksn = pl.pallas_call(
        _prep_sem_kernel,
        out_shape=jax.ShapeDtypeStruct((CAP_S, H), f32),
    )(sem_keys, sem_Wk)

    g_w2 = jnp.zeros((8, 64), f32).at[:3].set(gate_W2)
    g_b2 = jnp.zeros((1, 8), f32).at[0, :3].set(gate_b2)

    grid = (B // BQ,)
    full = lambda shape: pl.BlockSpec(shape, lambda i: (0, 0))
    out = pl.pallas_call(
        _main_kernel,
        grid=grid,
        in_specs=[
            pl.BlockSpec((BQ, H), lambda i: (i, 0)),
            full((CAP_E, H)),
            full((CAP_E, H)),
            full((1, CAP_E)),
            full((CAP_S, H)),
            full((CAP_S, H)),
            full((H, H)),
            full((H, H)),
            full((H, H)),
            full((64, H)),
            full((1, 64)),
            full((8, 64)),
            full((1, 8)),
            full((H, H)),
        ],
        out_specs=pl.BlockSpec((BQ, H), lambda i: (i, 0)),
        out_shape=jax.ShapeDtypeStruct((B, H), f32),
        compiler_params=pltpu.CompilerParams(
            dimension_semantics=("arbitrary",),
        ),
    )(query, kn, vals, w_row, ksn, sem_values, sem_Wq, ep_Wo, sem_Wo,
      gate_W1, gate_b1.reshape(1, 64), g_w2, g_b2, router_Wo)
    return out


# speedup vs baseline: 1.1579x; 1.1579x over previous
import functools

import jax
import jax.numpy as jnp
from jax.experimental import pallas as pl
from jax.experimental.pallas import tpu as pltpu

B = 4096
H = 128
CAP_E = 4096
CAP_S = 16384
TOPK_E = 8
TOPK_S = 4
RECENCY_DECAY = 0.99
TIMESTAMP = 100.0

BQ = 128


def _l2n(x):
    n = jnp.sqrt(jnp.sum(x * x, axis=-1, keepdims=True))
    return x / jnp.maximum(n, 1e-12)


def _prep_ep_kernel(store_ref, imp_ref, ts_ref, wk_ref, wv_ref,
                    kn_ref, vals_ref, w_ref):
    store = store_ref[...]
    keys = jnp.dot(store, wk_ref[...].T, preferred_element_type=jnp.float32)
    kn_ref[...] = _l2n(keys)
    ages = TIMESTAMP - ts_ref[...]
    recency = jnp.exp(-jnp.abs(ages) * (1.0 - RECENCY_DECAY))
    w = recency * (imp_ref[...] + 1.0)
    w_ref[...] = w / (jnp.sum(w) + 1e-8)
    vals_ref[...] = jnp.dot(store, wv_ref[...].T,
                            preferred_element_type=jnp.float32)


def _prep_sem_kernel(keys_ref, wk_ref, out_ref):
    k = jnp.dot(keys_ref[...], wk_ref[...].T,
                preferred_element_type=jnp.float32)
    out_ref[...] = _l2n(k)


def _rank_thresh(sim, k):
    t = jnp.max(sim, axis=-1, keepdims=True)
    for _ in range(k - 1):
        t = jnp.max(jnp.where(sim < t, sim, -jnp.inf),
                    axis=-1, keepdims=True)
    return t


def _masked_attn(sim, k):
    rowmax = jnp.max(sim, axis=-1, keepdims=True)
    ts = [rowmax]
    t = rowmax
    for _ in range(k - 1):
        t = jnp.max(jnp.where(sim < t, sim, -jnp.inf),
                    axis=-1, keepdims=True)
        ts.append(t)
    p = jnp.where(sim >= t, jnp.exp(sim - rowmax), 0.0)
    denom = jnp.ones_like(rowmax)
    for ti in ts[1:]:
        denom = denom + jnp.exp(ti - rowmax)
    return p, denom


def _main_kernel(q_ref, kn_ref, vals_ref, w_ref, ksn_ref, semv_ref,
                 sem_wq_ref, ep_wo_ref, sem_wo_ref,
                 g_w1_ref, g_b1_ref, g_w2_ref, g_b2_ref, r_wo_ref,
                 out_ref):
    q = q_ref[...]
    qn = _l2n(q)

    sim_e = jnp.dot(qn, kn_ref[...].T, preferred_element_type=jnp.float32)
    sim_e = sim_e * w_ref[...]
    we, den_e = _masked_attn(sim_e, TOPK_E)
    ep_out = jnp.dot(we, vals_ref[...], preferred_element_type=jnp.float32)
    ep_out = ep_out / den_e
    ep_out = jnp.dot(ep_out, ep_wo_ref[...].T,
                     preferred_element_type=jnp.float32)

    qs = jnp.dot(q, sem_wq_ref[...].T, preferred_element_type=jnp.float32)
    qsn = _l2n(qs)
    sim_s = jnp.dot(qsn, ksn_ref[...].T, preferred_element_type=jnp.float32)
    ws, den_s = _masked_attn(sim_s, TOPK_S)
    sem_out = jnp.dot(ws, semv_ref[...], preferred_element_type=jnp.float32)
    sem_out = sem_out / den_s
    sem_out = jnp.dot(sem_out, sem_wo_ref[...].T,
                      preferred_element_type=jnp.float32)

    h = jnp.dot(q, g_w1_ref[...].T, preferred_element_type=jnp.float32)
    h = h + g_b1_ref[...]
    h = h * jax.nn.sigmoid(h)
    logits = jnp.dot(h, g_w2_ref[...].T, preferred_element_type=jnp.float32)
    logits = logits + g_b2_ref[...]
    col = jax.lax.broadcasted_iota(jnp.int32, logits.shape, 1)
    logits = jnp.where(col < 3, logits, -jnp.inf)
    gmax = jnp.max(logits, axis=-1, keepdims=True)
    ge = jnp.exp(logits - gmax)
    gw = ge / jnp.sum(ge, axis=-1, keepdims=True)

    blended = (gw[:, 0:1] * q + gw[:, 1:2] * ep_out + gw[:, 2:3] * sem_out)
    out_ref[...] = jnp.dot(blended, r_wo_ref[...].T,
                           preferred_element_type=jnp.float32)


@jax.jit
def kernel(query, ep_store, ep_importance, ep_timestamps, ep_Wk, ep_Wv, ep_Wo,
           sem_keys, sem_values, sem_Wq, sem_Wk, sem_Wo,
           gate_W1, gate_b1, gate_W2, gate_b2, router_Wo):
    f32 = jnp.float32

    kn, vals, w_row = pl.pallas_call(
        _prep_ep_kernel,
        out_shape=(jax.ShapeDtypeStruct((CAP_E, H), f32),
                   jax.ShapeDtypeStruct((CAP_E, H), f32),
                   jax.ShapeDtypeStruct((1, CAP_E), f32)),
    )(ep_store, ep_importance.reshape(1, CAP_E),
      ep_timestamps.reshape(1, CAP_E), ep_Wk, ep_Wv)

    ksn = pl.pallas_call(
        _prep_sem_kernel,
        out_shape=jax.ShapeDtypeStruct((CAP_S, H), f32),
    )(sem_keys, sem_Wk)

    g_w2 = jnp.zeros((8, 64), f32).at[:3].set(gate_W2)
    g_b2 = jnp.zeros((1, 8), f32).at[0, :3].set(gate_b2)

    grid = (B // BQ,)
    full = lambda shape: pl.BlockSpec(shape, lambda i: (0, 0))
    out = pl.pallas_call(
        _main_kernel,
        grid=grid,
        in_specs=[
            pl.BlockSpec((BQ, H), lambda i: (i, 0)),
            full((CAP_E, H)),
            full((CAP_E, H)),
            full((1, CAP_E)),
            full((CAP_S, H)),
            full((CAP_S, H)),
            full((H, H)),
            full((H, H)),
            full((H, H)),
            full((64, H)),
            full((1, 64)),
            full((8, 64)),
            full((1, 8)),
            full((H, H)),
        ],
        out_specs=pl.BlockSpec((BQ, H), lambda i: (i, 0)),
        out_shape=jax.ShapeDtypeStruct((B, H), f32),
        compiler_params=pltpu.CompilerParams(
            dimension_semantics=("arbitrary",),
        ),
    )(query, kn, vals, w_row, ksn, sem_values, sem_Wq, ep_Wo, sem_Wo,
      gate_W1, gate_b1.reshape(1, 64), g_w2, g_b2, router_Wo)
    return out


# speedup vs baseline: 1.3496x; 1.1655x over previous
import functools

import jax
import jax.numpy as jnp
from jax.experimental import pallas as pl
from jax.experimental.pallas import tpu as pltpu

B = 4096
H = 128
CAP_E = 4096
CAP_S = 16384
TOPK_E = 8
TOPK_S = 4
RECENCY_DECAY = 0.99
TIMESTAMP = 100.0

BQ = 128


def _l2n(x):
    n = jnp.sqrt(jnp.sum(x * x, axis=-1, keepdims=True))
    return x / jnp.maximum(n, 1e-12)


def _prep_ep_kernel(store_ref, imp_ref, ts_ref, wk_ref, wv_ref,
                    kn_ref, vals_ref, w_ref):
    store = store_ref[...]
    keys = jnp.dot(store, wk_ref[...].T, preferred_element_type=jnp.float32)
    kn_ref[...] = _l2n(keys)
    ages = TIMESTAMP - ts_ref[...]
    recency = jnp.exp(-jnp.abs(ages) * (1.0 - RECENCY_DECAY))
    w = recency * (imp_ref[...] + 1.0)
    w_ref[...] = w / (jnp.sum(w) + 1e-8)
    vals_ref[...] = jnp.dot(store, wv_ref[...].T,
                            preferred_element_type=jnp.float32)


def _prep_sem_kernel(keys_ref, wk_ref, out_ref):
    k = jnp.dot(keys_ref[...], wk_ref[...].T,
                preferred_element_type=jnp.float32)
    out_ref[...] = _l2n(k)


def _rank_thresh(sim, k):
    t = jnp.max(sim, axis=-1, keepdims=True)
    for _ in range(k - 1):
        t = jnp.max(jnp.where(sim < t, sim, -jnp.inf),
                    axis=-1, keepdims=True)
    return t


def _masked_attn(sim, k):
    chunk = 128
    nchunks = sim.shape[-1] // chunk
    regs = [jnp.full((sim.shape[0], chunk), -jnp.inf, sim.dtype)] * k
    for i in range(nchunks):
        x = sim[:, i * chunk:(i + 1) * chunk]
        nr = []
        for j in range(k):
            hi = jnp.maximum(regs[j], x)
            if j < k - 1:
                x = jnp.minimum(regs[j], x)
            nr.append(hi)
        regs = nr
    cand = jnp.concatenate(regs, axis=-1)
    rowmax = jnp.max(cand, axis=-1, keepdims=True)
    ts = [rowmax]
    t = rowmax
    for _ in range(k - 1):
        t = jnp.max(jnp.where(cand < t, cand, -jnp.inf),
                    axis=-1, keepdims=True)
        ts.append(t)
    p = jnp.where(sim >= t, jnp.exp(sim - rowmax), 0.0)
    denom = jnp.ones_like(rowmax)
    for ti in ts[1:]:
        denom = denom + jnp.exp(ti - rowmax)
    return p, denom


def _main_kernel(q_ref, kn_ref, vals_ref, w_ref, ksn_ref, semv_ref,
                 sem_wq_ref, ep_wo_ref, sem_wo_ref,
                 g_w1_ref, g_b1_ref, g_w2_ref, g_b2_ref, r_wo_ref,
                 out_ref):
    q = q_ref[...]
    qn = _l2n(q)

    sim_e = jnp.dot(qn, kn_ref[...].T, preferred_element_type=jnp.float32)
    sim_e = sim_e * w_ref[...]
    we, den_e = _masked_attn(sim_e, TOPK_E)
    ep_out = jnp.dot(we, vals_ref[...], preferred_element_type=jnp.float32)
    ep_out = ep_out / den_e
    ep_out = jnp.dot(ep_out, ep_wo_ref[...].T,
                     preferred_element_type=jnp.float32)

    qs = jnp.dot(q, sem_wq_ref[...].T, preferred_element_type=jnp.float32)
    qsn = _l2n(qs)
    sim_s = jnp.dot(qsn, ksn_ref[...].T, preferred_element_type=jnp.float32)
    ws, den_s = _masked_attn(sim_s, TOPK_S)
    sem_out = jnp.dot(ws, semv_ref[...], preferred_element_type=jnp.float32)
    sem_out = sem_out / den_s
    sem_out = jnp.dot(sem_out, sem_wo_ref[...].T,
                      preferred_element_type=jnp.float32)

    h = jnp.dot(q, g_w1_ref[...].T, preferred_element_type=jnp.float32)
    h = h + g_b1_ref[...]
    h = h * jax.nn.sigmoid(h)
    logits = jnp.dot(h, g_w2_ref[...].T, preferred_element_type=jnp.float32)
    logits = logits + g_b2_ref[...]
    col = jax.lax.broadcasted_iota(jnp.int32, logits.shape, 1)
    logits = jnp.where(col < 3, logits, -jnp.inf)
    gmax = jnp.max(logits, axis=-1, keepdims=True)
    ge = jnp.exp(logits - gmax)
    gw = ge / jnp.sum(ge, axis=-1, keepdims=True)

    blended = (gw[:, 0:1] * q + gw[:, 1:2] * ep_out + gw[:, 2:3] * sem_out)
    out_ref[...] = jnp.dot(blended, r_wo_ref[...].T,
                           preferred_element_type=jnp.float32)


@jax.jit
def kernel(query, ep_store, ep_importance, ep_timestamps, ep_Wk, ep_Wv, ep_Wo,
           sem_keys, sem_values, sem_Wq, sem_Wk, sem_Wo,
           gate_W1, gate_b1, gate_W2, gate_b2, router_Wo):
    f32 = jnp.float32

    kn, vals, w_row = pl.pallas_call(
        _prep_ep_kernel,
        out_shape=(jax.ShapeDtypeStruct((CAP_E, H), f32),
                   jax.ShapeDtypeStruct((CAP_E, H), f32),
                   jax.ShapeDtypeStruct((1, CAP_E), f32)),
    )(ep_store, ep_importance.reshape(1, CAP_E),
      ep_timestamps.reshape(1, CAP_E), ep_Wk, ep_Wv)

    ksn = pl.pallas_call(
        _prep_sem_kernel,
        out_shape=jax.ShapeDtypeStruct((CAP_S, H), f32),
    )(sem_keys, sem_Wk)

    g_w2 = jnp.zeros((8, 64), f32).at[:3].set(gate_W2)
    g_b2 = jnp.zeros((1, 8), f32).at[0, :3].set(gate_b2)

    grid = (B // BQ,)
    full = lambda shape: pl.BlockSpec(shape, lambda i: (0, 0))
    out = pl.pallas_call(
        _main_kernel,
        grid=grid,
        in_specs=[
            pl.BlockSpec((BQ, H), lambda i: (i, 0)),
            full((CAP_E, H)),
            full((CAP_E, H)),
            full((1, CAP_E)),
            full((CAP_S, H)),
            full((CAP_S, H)),
            full((H, H)),
            full((H, H)),
            full((H, H)),
            full((64, H)),
            full((1, 64)),
            full((8, 64)),
            full((1, 8)),
            full((H, H)),
        ],
        out_specs=pl.BlockSpec((BQ, H), lambda i: (i, 0)),
        out_shape=jax.ShapeDtypeStruct((B, H), f32),
        compiler_params=pltpu.CompilerParams(
            dimension_semantics=("arbitrary",),
        ),
    )(query, kn, vals, w_row, ksn, sem_values, sem_Wq, ep_Wo, sem_Wo,
      gate_W1, gate_b1.reshape(1, 64), g_w2, g_b2, router_Wo)
    return out
